# baseline (speedup 1.0000x reference)
import functools

import numpy as np
import jax
import jax.numpy as jnp
from jax.experimental import pallas as pl
from jax.experimental.pallas import tpu as pltpu

_D = 512
_CS = (8, 16, 128)


def _windows(L):
    t = np.arange(_D)
    starts = (t * L) // _D
    ends = -((-(t + 1) * L) // _D)
    return starts, ends


@functools.lru_cache(maxsize=None)
def _pool_tables():
    masks, invs = [], []
    for C in _CS[:2]:
        L = 9 * C
        starts, ends = _windows(L)
        l = np.arange(L)[:, None]
        mask = ((l >= starts[None, :]) & (l < ends[None, :])).astype(np.float32)
        perm = (np.arange(L) % C) * 9 + (np.arange(L) // C)
        masks.append(mask[perm, :])
        invs.append(1.0 / (ends - starts).astype(np.float32))
    C = _CS[2]
    L = 9 * C
    starts, ends = _windows(L)
    l = np.arange(L)[:, None]
    mask2 = ((l >= starts[None, :]) & (l < ends[None, :])).astype(np.float32)
    p2 = np.empty((9, C, _D), dtype=np.float32)
    for j in range(9):
        p2[j] = mask2[np.arange(C) * 9 + j, :]
    masks.append(p2)
    invs.append(1.0 / (ends - starts).astype(np.float32))
    inv_row = np.concatenate(invs)[None, :]
    return tuple(masks), inv_row


_CONTRACT_0_0 = (((0,), (0,)), ((), ()))


def _fused_pool_kernel(x0_ref, x1_ref, x2_ref, p0_ref, p1_ref, p2_ref,
                       inv_ref, o_ref):
    s = inv_ref[...]
    a0 = jax.lax.dot_general(x0_ref[...].astype(jnp.bfloat16), p0_ref[...],
                             _CONTRACT_0_0,
                             preferred_element_type=jnp.float32)
    o_ref[0] = a0 * s[:, 0 * _D:1 * _D]
    a1 = jax.lax.dot_general(x1_ref[...].astype(jnp.bfloat16), p1_ref[...],
                             _CONTRACT_0_0,
                             preferred_element_type=jnp.float32)
    o_ref[1] = a1 * s[:, 1 * _D:2 * _D]
    a2 = jnp.dot(x2_ref[0].astype(jnp.bfloat16), p2_ref[0],
                 preferred_element_type=jnp.float32)
    for j in range(1, 9):
        a2 += jnp.dot(x2_ref[j].astype(jnp.bfloat16), p2_ref[j],
                      preferred_element_type=jnp.float32)
    o_ref[2] = a2 * s[:, 2 * _D:3 * _D]


def kernel(feat0, feat1, feat2):
    B = feat0.shape[0]
    x0 = feat0.transpose(2, 3, 1, 0).reshape(9 * feat0.shape[1], B)
    x1 = feat1.transpose(2, 3, 1, 0).reshape(9 * feat1.shape[1], B)
    x2 = feat2.transpose(2, 3, 0, 1).reshape(9, B, feat2.shape[1])
    masks_np, inv_np = _pool_tables()
    ps = [jnp.asarray(m, dtype=jnp.bfloat16) for m in masks_np]
    inv = jnp.asarray(inv_np)

    TB = 256
    grid = (B // TB,)
    ls = [9 * c for c in _CS]

    flops = 2 * B * sum(ls) * _D
    bytes_accessed = 4 * (B * sum(ls) + B * 3 * _D) + 2 * sum(ls) * _D
    cost = pl.CostEstimate(flops=flops, transcendentals=0,
                           bytes_accessed=bytes_accessed)

    out = pl.pallas_call(
        _fused_pool_kernel,
        out_shape=jax.ShapeDtypeStruct((3, B, _D), jnp.float32),
        grid=grid,
        in_specs=[
            pl.BlockSpec((ls[0], TB), lambda i: (0, i)),
            pl.BlockSpec((ls[1], TB), lambda i: (0, i)),
            pl.BlockSpec((9, TB, _CS[2]), lambda i: (0, i, 0)),
            pl.BlockSpec((ls[0], _D), lambda i: (0, 0)),
            pl.BlockSpec((ls[1], _D), lambda i: (0, 0)),
            pl.BlockSpec((9, _CS[2], _D), lambda i: (0, 0, 0)),
            pl.BlockSpec((1, 3 * _D), lambda i: (0, 0)),
        ],
        out_specs=pl.BlockSpec((3, TB, _D), lambda i: (0, i, 0)),
        compiler_params=pltpu.CompilerParams(
            dimension_semantics=("parallel",)),
        cost_estimate=cost,
    )(x0, x1, x2, ps[0], ps[1], ps[2], inv)

    return out.transpose(1, 0, 2)


# speedup vs baseline: 1.3569x; 1.3569x over previous
import functools

import numpy as np
import jax
import jax.numpy as jnp
from jax.experimental import pallas as pl
from jax.experimental.pallas import tpu as pltpu

_D = 512
_CS = (8, 16, 128)


def _windows(L):
    t = np.arange(_D)
    starts = (t * L) // _D
    ends = -((-(t + 1) * L) // _D)
    return starts, ends


@functools.lru_cache(maxsize=None)
def _pool_tables():
    masks, invs = [], []
    for C in _CS[:2]:
        L = 9 * C
        starts, ends = _windows(L)
        l = np.arange(L)[:, None]
        mask = ((l >= starts[None, :]) & (l < ends[None, :])).astype(np.float32)
        perm = (np.arange(L) % C) * 9 + (np.arange(L) // C)
        masks.append(mask[perm, :])
        invs.append(1.0 / (ends - starts).astype(np.float32))
    C = _CS[2]
    L = 9 * C
    starts, ends = _windows(L)
    l = np.arange(L)[:, None]
    mask2 = ((l >= starts[None, :]) & (l < ends[None, :])).astype(np.float32)
    p2 = np.empty((9, C, _D), dtype=np.float32)
    for j in range(9):
        p2[j] = mask2[np.arange(C) * 9 + j, :]
    masks.append(p2)
    invs.append(1.0 / (ends - starts).astype(np.float32))
    inv_row = np.concatenate(invs)[None, :]
    return tuple(masks), inv_row


_CONTRACT_0_0 = (((0,), (0,)), ((), ()))


def _fused_pool_kernel(x0_ref, x1_ref, x2_ref, p0_ref, p1_ref, p2_ref,
                       inv_ref, o_ref):
    s = inv_ref[...]
    a0 = jax.lax.dot_general(x0_ref[...].astype(jnp.bfloat16), p0_ref[...],
                             _CONTRACT_0_0,
                             preferred_element_type=jnp.float32)
    o_ref[0] = a0 * s[:, 0 * _D:1 * _D]
    a1 = jax.lax.dot_general(x1_ref[...].astype(jnp.bfloat16), p1_ref[...],
                             _CONTRACT_0_0,
                             preferred_element_type=jnp.float32)
    o_ref[1] = a1 * s[:, 1 * _D:2 * _D]
    a2 = jnp.dot(x2_ref[0].astype(jnp.bfloat16), p2_ref[0],
                 preferred_element_type=jnp.float32)
    for j in range(1, 9):
        a2 += jnp.dot(x2_ref[j].astype(jnp.bfloat16), p2_ref[j],
                      preferred_element_type=jnp.float32)
    o_ref[2] = a2 * s[:, 2 * _D:3 * _D]


def kernel(feat0, feat1, feat2):
    B = feat0.shape[0]
    x0 = feat0.transpose(2, 3, 1, 0).reshape(9 * feat0.shape[1], B)
    x1 = feat1.transpose(2, 3, 1, 0).reshape(9 * feat1.shape[1], B)
    x2 = feat2.transpose(2, 3, 0, 1).reshape(9, B, feat2.shape[1])
    masks_np, inv_np = _pool_tables()
    ps = [jnp.asarray(m, dtype=jnp.bfloat16) for m in masks_np]
    inv = jnp.asarray(inv_np)

    TB = 1024
    grid = (B // TB,)
    ls = [9 * c for c in _CS]

    flops = 2 * B * sum(ls) * _D
    bytes_accessed = 4 * (B * sum(ls) + B * 3 * _D) + 2 * sum(ls) * _D
    cost = pl.CostEstimate(flops=flops, transcendentals=0,
                           bytes_accessed=bytes_accessed)

    out = pl.pallas_call(
        _fused_pool_kernel,
        out_shape=jax.ShapeDtypeStruct((3, B, _D), jnp.float32),
        grid=grid,
        in_specs=[
            pl.BlockSpec((ls[0], TB), lambda i: (0, i)),
            pl.BlockSpec((ls[1], TB), lambda i: (0, i)),
            pl.BlockSpec((9, TB, _CS[2]), lambda i: (0, i, 0)),
            pl.BlockSpec((ls[0], _D), lambda i: (0, 0)),
            pl.BlockSpec((ls[1], _D), lambda i: (0, 0)),
            pl.BlockSpec((9, _CS[2], _D), lambda i: (0, 0, 0)),
            pl.BlockSpec((1, 3 * _D), lambda i: (0, 0)),
        ],
        out_specs=pl.BlockSpec((3, TB, _D), lambda i: (0, i, 0)),
        compiler_params=pltpu.CompilerParams(
            dimension_semantics=("parallel",)),
        cost_estimate=cost,
    )(x0, x1, x2, ps[0], ps[1], ps[2], inv)

    return out.transpose(1, 0, 2)


# speedup vs baseline: 1.3615x; 1.0034x over previous
import functools

import numpy as np
import jax
import jax.numpy as jnp
from jax.experimental import pallas as pl
from jax.experimental.pallas import tpu as pltpu

_D = 512
_CS = (8, 16, 128)


def _windows(L):
    t = np.arange(_D)
    starts = (t * L) // _D
    ends = -((-(t + 1) * L) // _D)
    return starts, ends


@functools.lru_cache(maxsize=None)
def _pool_tables():
    masks, invs = [], []
    for C in _CS[:2]:
        L = 9 * C
        starts, ends = _windows(L)
        l = np.arange(L)[:, None]
        mask = ((l >= starts[None, :]) & (l < ends[None, :])).astype(np.float32)
        perm = (np.arange(L) % C) * 9 + (np.arange(L) // C)
        masks.append(mask[perm, :])
        invs.append(1.0 / (ends - starts).astype(np.float32))
    C = _CS[2]
    L = 9 * C
    starts, ends = _windows(L)
    l = np.arange(L)[:, None]
    mask2 = ((l >= starts[None, :]) & (l < ends[None, :])).astype(np.float32)
    p2 = np.empty((9, C, _D), dtype=np.float32)
    for j in range(9):
        p2[j] = mask2[np.arange(C) * 9 + j, :]
    masks.append(p2)
    invs.append(1.0 / (ends - starts).astype(np.float32))
    inv_row = np.concatenate(invs)[None, :]
    return tuple(masks), inv_row


_CONTRACT_0_0 = (((0,), (0,)), ((), ()))


def _fused_pool_kernel(x0_ref, x1_ref, x2_ref, p0_ref, p1_ref, p2_ref,
                       inv_ref, o_ref):
    s = inv_ref[...]
    a0 = jax.lax.dot_general(x0_ref[...].astype(jnp.bfloat16), p0_ref[...],
                             _CONTRACT_0_0,
                             preferred_element_type=jnp.float32)
    o_ref[0] = a0 * s[:, 0 * _D:1 * _D]
    a1 = jax.lax.dot_general(x1_ref[...].astype(jnp.bfloat16), p1_ref[...],
                             _CONTRACT_0_0,
                             preferred_element_type=jnp.float32)
    o_ref[1] = a1 * s[:, 1 * _D:2 * _D]
    a2 = jnp.dot(x2_ref[0].astype(jnp.bfloat16), p2_ref[0],
                 preferred_element_type=jnp.float32)
    for j in range(1, 9):
        a2 += jnp.dot(x2_ref[j].astype(jnp.bfloat16), p2_ref[j],
                      preferred_element_type=jnp.float32)
    o_ref[2] = a2 * s[:, 2 * _D:3 * _D]


def kernel(feat0, feat1, feat2):
    B = feat0.shape[0]
    x0 = feat0.transpose(2, 3, 1, 0).reshape(9 * feat0.shape[1], B)
    x1 = feat1.transpose(2, 3, 1, 0).reshape(9 * feat1.shape[1], B)
    x2 = feat2.transpose(2, 3, 0, 1).reshape(9, B, feat2.shape[1])
    masks_np, inv_np = _pool_tables()
    ps = [jnp.asarray(m, dtype=jnp.bfloat16) for m in masks_np]
    inv = jnp.asarray(inv_np)

    TB = 1024
    while B % TB:
        TB //= 2
    grid = (B // TB,)
    ls = [9 * c for c in _CS]

    flops = 2 * B * sum(ls) * _D
    bytes_accessed = 4 * (B * sum(ls) + B * 3 * _D) + 2 * sum(ls) * _D
    cost = pl.CostEstimate(flops=flops, transcendentals=0,
                           bytes_accessed=bytes_accessed)

    out = pl.pallas_call(
        _fused_pool_kernel,
        out_shape=jax.ShapeDtypeStruct((3, B, _D), jnp.float32),
        grid=grid,
        in_specs=[
            pl.BlockSpec((ls[0], TB), lambda i: (0, i)),
            pl.BlockSpec((ls[1], TB), lambda i: (0, i)),
            pl.BlockSpec((9, TB, _CS[2]), lambda i: (0, i, 0)),
            pl.BlockSpec((ls[0], _D), lambda i: (0, 0)),
            pl.BlockSpec((ls[1], _D), lambda i: (0, 0)),
            pl.BlockSpec((9, _CS[2], _D), lambda i: (0, 0, 0)),
            pl.BlockSpec((1, 3 * _D), lambda i: (0, 0)),
        ],
        out_specs=pl.BlockSpec((3, TB, _D), lambda i: (0, i, 0)),
        compiler_params=pltpu.CompilerParams(
            dimension_semantics=("parallel",)),
        cost_estimate=cost,
    )(x0, x1, x2, ps[0], ps[1], ps[2], inv)

    return out.transpose(1, 0, 2)
